# baseline (speedup 1.0000x reference)
import jax
import jax.numpy as jnp
from jax import lax
from jax.experimental import pallas as pl
from jax.experimental.pallas import tpu as pltpu
from jax.experimental.pallas import tpu_sc as plsc

_S = 200
_B = 4096
_D = 64
_V = 1000000
_N = _S * _B
_L = 16
_NQ = _D // _L
_NC = 2
_NS = 16
_NW = _NC * _NS

_NSLAB = _V // 128
_SLAB_PER_W = (_NSLAB + _NW - 1) // _NW

_C = 256
_SUB = 128
_NSUB = _C // _SUB
_NCHUNK = _N // _C
_CPS = _B // _C
_PER_W = _NCHUNK // _NW
_SLABW = _D * _B
_CHW = _C * _D
_TI = _CHW // 8


def _rot_vecs():
    lane = lax.iota(jnp.int32, _L)
    return [(lane + r) & (_L - 1) for r in range(_L)]


def _relayout_body(tbt_hbm, tail_hbm, out_hbm, slab_v, dst_v, skew_v, sem, osem):
    wid = lax.axis_index("s") * _NC + lax.axis_index("c")
    rot = _rot_vecs()
    lane128 = lax.iota(jnp.int32, _L) * 128

    j0 = wid * _SLAB_PER_W
    nslab_w = jnp.minimum(_SLAB_PER_W, _NSLAB - j0)

    def stage(t, buf):
        j = j0 + t
        pltpu.async_copy(
            tbt_hbm.at[pl.ds(0, _D), pl.ds(j * 128, 128)], slab_v.at[buf], sem
        )

    def wait_stage(t, buf):
        j = j0 + t
        pltpu.make_async_copy(
            tbt_hbm.at[pl.ds(0, _D), pl.ds(j * 128, 128)], slab_v.at[buf], sem
        ).wait()

    stage(0, 0)

    def slab_body(t, carry):
        j = j0 + t
        buf = t % 2

        @pl.when(t + 1 < nslab_w)
        def _():
            stage(t + 1, (t + 1) % 2)

        wait_stage(t, buf)

        def skew_d16(dd, c2):
            for d0 in range(_L):
                for k in range(8):
                    v = slab_v[buf, dd * _L + d0, pl.ds(k * _L, _L)]
                    idx = (dd * _L + d0) * 128 + k * _L + rot[d0]
                    plsc.store_scatter(skew_v, [idx], v)
            return c2

        lax.fori_loop(0, _NQ, skew_d16, 0)

        def rd_t16(t16, c2):
            for t0 in range(_L):
                for q in range(_NQ):
                    idx = lane128 + rot[t0] + (q * _L * 128 + t16 * _L)
                    v = plsc.load_gather(skew_v, [idx])
                    dst_v[buf, 8 * t16 + t0 // 2, pl.ds((t0 % 2) * _D + q * _L, _L)] = v
            return c2

        lax.fori_loop(0, 8, rd_t16, 0)

        @pl.when(t >= 2)
        def _():
            pltpu.make_async_copy(
                dst_v.at[buf], out_hbm.at[pl.ds((j - 2) * _D, _D)], osem
            ).wait()

        pltpu.async_copy(dst_v.at[buf], out_hbm.at[pl.ds(j * _D, _D)], osem)
        return carry

    lax.fori_loop(0, nslab_w, slab_body, 0)

    def drain(t):
        j = j0 + t
        pltpu.make_async_copy(
            dst_v.at[t % 2], out_hbm.at[pl.ds(j * _D, _D)], osem
        ).wait()

    drain(nslab_w - 2)
    drain(nslab_w - 1)

    @pl.when(wid == _NW - 1)
    def _():
        pltpu.sync_copy(tail_hbm, slab_v.at[0])

        def tskew(dd, c2):
            for d0 in range(_L):
                for k in range(8):
                    v = slab_v[0, dd * _L + d0, pl.ds(k * _L, _L)]
                    idx = (dd * _L + d0) * 128 + k * _L + rot[d0]
                    plsc.store_scatter(skew_v, [idx], v)
            return c2

        lax.fori_loop(0, _NQ, tskew, 0)

        def trd(t16, c2):
            for t0 in range(_L):
                for q in range(_NQ):
                    idx = lane128 + rot[t0] + (q * _L * 128 + t16 * _L)
                    v = plsc.load_gather(skew_v, [idx])
                    dst_v[0, 8 * t16 + t0 // 2, pl.ds((t0 % 2) * _D + q * _L, _L)] = v
            return c2

        lax.fori_loop(0, 8, trd, 0)
        pltpu.sync_copy(dst_v.at[0], out_hbm.at[pl.ds(_NSLAB * _D, _D)])


def _lookup_body(x_hbm, table_hbm, pos_hbm, out_hbm,
                 idx_v, rows_v, skew_v, outt_v, pos_v, gsem, osem):
    wid = lax.axis_index("s") * _NC + lax.axis_index("c")
    rot = _rot_vecs()
    lane64 = lax.iota(jnp.int32, _L) * _D
    pltpu.sync_copy(pos_hbm.at[pl.ds(0, _S)], pos_v)

    def stage(t, buf):
        g = wid * _PER_W + t
        pltpu.sync_copy(x_hbm.at[pl.ds(g * _NSUB, _NSUB)], idx_v.at[buf])
        for j in range(_NSUB):
            pltpu.async_copy(
                table_hbm.at[idx_v.at[buf, j]],
                rows_v.at[buf, pl.ds(j * _SUB, _SUB)],
                gsem,
            )

    def wait_gather(buf):
        for j in range(_NSUB):
            pltpu.make_async_copy(
                table_hbm.at[idx_v.at[buf, j]],
                rows_v.at[buf, pl.ds(j * _SUB, _SUB)],
                gsem,
            ).wait()

    def out_slices(g, buf):
        s_idx = g // _CPS
        j0 = g % _CPS
        return [
            (
                outt_v.at[buf, pl.ds(i * _TI, _TI)],
                out_hbm.at[s_idx, pl.ds(i * (_CPS * _TI) + j0 * _TI, _TI)],
            )
            for i in range(8)
        ]

    stage(0, 0)

    def chunk_body(t, carry):
        g = wid * _PER_W + t
        buf = t % 2

        @pl.when(t + 1 < _PER_W)
        def _():
            stage(t + 1, (t + 1) % 2)

        wait_gather(buf)

        s_idx = g // _CPS
        pos_regs = [pos_v[s_idx, pl.ds(q * _L, _L)] for q in range(_NQ)]

        @pl.when(t >= 2)
        def _():
            for src, dst in out_slices(g - 2, buf):
                pltpu.make_async_copy(src, dst, osem).wait()

        def skew_t16(t16, c2):
            tb = t16 * _L
            for t0 in range(_L):
                tt = tb + t0
                for q in range(_NQ):
                    v = rows_v[buf, tt, pl.ds(q * _L, _L)] + pos_regs[q]
                    idx = tt * _D + q * _L + rot[t0]
                    plsc.store_scatter(skew_v, [idx], v)
            return c2

        lax.fori_loop(0, _C // _L, skew_t16, 0)

        for d in range(_D):
            cvec = lane64 + rot[d % _L] + (d // _L) * _L

            def rd_tb(t16, c3, cvec=cvec, d=d):
                tb = t16 * _L
                idx = cvec + tb * _D
                v = plsc.load_gather(skew_v, [idx])
                off = (d // 8) * _TI + (tb // _SUB) * (8 * _SUB) \
                    + (d % 8) * _SUB + (tb % _SUB)
                outt_v[buf, pl.ds(off, _L)] = v
                return c3

            lax.fori_loop(0, _C // _L, rd_tb, 0)

        for src, dst in out_slices(g, buf):
            pltpu.async_copy(src, dst, osem)
        return carry

    lax.fori_loop(0, _PER_W, chunk_body, 0)

    for tail in (_PER_W - 2, _PER_W - 1):
        g = wid * _PER_W + tail
        for src, dst in out_slices(g, tail % 2):
            pltpu.make_async_copy(src, dst, osem).wait()


@jax.jit
def _run(x, embed_table, pos_table):
    mesh = plsc.VectorSubcoreMesh(core_axis_name="c", subcore_axis_name="s")
    x2d = x.reshape(_N // _SUB, _SUB)
    tbt = embed_table.T
    tbt_tail = jnp.pad(tbt[:, _NSLAB * 128:], ((0, 0), (0, 64)))

    tlin = pl.kernel(
        _relayout_body,
        out_type=jax.ShapeDtypeStruct((_V // 2 + 32, 128), jnp.float32),
        mesh=mesh,
        scratch_types=[
            pltpu.VMEM((2, _D, 128), jnp.float32),
            pltpu.VMEM((2, _D, 128), jnp.float32),
            pltpu.VMEM((_D * 128,), jnp.float32),
            pltpu.SemaphoreType.DMA,
            pltpu.SemaphoreType.DMA,
        ],
        compiler_params=pltpu.CompilerParams(
            use_tc_tiling_on_sc=True, needs_layout_passes=False
        ),
    )(tbt, tbt_tail)

    table_lin = tlin.reshape(_V + _D, _D)

    out = pl.kernel(
        _lookup_body,
        out_type=jax.ShapeDtypeStruct((_S, _SLABW), jnp.float32),
        mesh=mesh,
        scratch_types=[
            pltpu.VMEM((2, _NSUB, _SUB), jnp.int32),
            pltpu.VMEM((2, _C, _D), jnp.float32),
            pltpu.VMEM((_CHW,), jnp.float32),
            pltpu.VMEM((2, _CHW), jnp.float32),
            pltpu.VMEM((_S, _D), jnp.float32),
            pltpu.SemaphoreType.DMA,
            pltpu.SemaphoreType.DMA,
        ],
        compiler_params=pltpu.CompilerParams(
            use_tc_tiling_on_sc=False, needs_layout_passes=False
        ),
    )(x2d, table_lin, pos_table)
    out = out.reshape(_S, 8, _B // _SUB, 8, _SUB)
    out = out.transpose(0, 1, 3, 2, 4).reshape(_S, _D, _B).transpose(0, 2, 1)
    return out


def kernel(x, embed_table, pos_table):
    return _run(x, embed_table, pos_table)


# speedup vs baseline: 1.7483x; 1.7483x over previous
import jax
import jax.numpy as jnp
from jax import lax
from jax.experimental import pallas as pl
from jax.experimental.pallas import tpu as pltpu
from jax.experimental.pallas import tpu_sc as plsc

_S = 200
_B = 4096
_D = 64
_C = 512
_SUB = 128
_NSUB = _C // _SUB
_N = _S * _B
_NCHUNK = _N // _C
_CPS = _B // _C
_NC = 2
_NS = 16
_NW = _NC * _NS
_PER_W = _NCHUNK // _NW
_L = 16
_NQ = _D // _L


def _emb_body(x_hbm, table_hbm, pos_hbm, out_hbm, idx_v, rows_v, pos_v, gsem, osem):
    wid = lax.axis_index("s") * _NC + lax.axis_index("c")
    pltpu.sync_copy(pos_hbm.at[pl.ds(0, _S)], pos_v)

    def stage_and_fire(t, buf):
        g = wid * _PER_W + t
        pltpu.sync_copy(x_hbm.at[pl.ds(g * _NSUB, _NSUB)], idx_v.at[buf])
        for j in range(_NSUB):
            pltpu.async_copy(
                table_hbm.at[idx_v.at[buf, j]],
                rows_v.at[buf, pl.ds(j * _SUB, _SUB)],
                gsem,
            )

    def drain_gather(buf):
        for j in range(_NSUB):
            pltpu.make_async_copy(
                table_hbm.at[idx_v.at[buf, j]],
                rows_v.at[buf, pl.ds(j * _SUB, _SUB)],
                gsem,
            ).wait()

    stage_and_fire(0, 0)

    def chunk_body(t, carry):
        g = wid * _PER_W + t
        buf = t % 2

        @pl.when(t + 1 < _PER_W)
        def _():
            stage_and_fire(t + 1, (t + 1) % 2)

        drain_gather(buf)

        s_idx = g // _CPS
        pos_regs = [pos_v[s_idx, pl.ds(q * _L, _L)] for q in range(_NQ)]

        def row_body(i, c2):
            for q in range(_NQ):
                plsc.addupdate(rows_v.at[buf, i, pl.ds(q * _L, _L)], pos_regs[q])
            return c2

        lax.fori_loop(0, _C, row_body, 0, unroll=8)

        @pl.when(t >= 2)
        def _():
            pltpu.make_async_copy(
                rows_v.at[buf],
                out_hbm.at[pl.ds((g - 2) * _C, _C)],
                osem,
            ).wait()

        pltpu.async_copy(rows_v.at[buf], out_hbm.at[pl.ds(g * _C, _C)], osem)
        return carry

    lax.fori_loop(0, _PER_W, chunk_body, 0)

    for tail in (_PER_W - 2, _PER_W - 1):
        g = wid * _PER_W + tail
        pltpu.make_async_copy(
            rows_v.at[tail % 2],
            out_hbm.at[pl.ds(g * _C, _C)],
            osem,
        ).wait()


@jax.jit
def _run(x, embed_table, pos_table):
    mesh = plsc.VectorSubcoreMesh(core_axis_name="c", subcore_axis_name="s")
    x2d = x.reshape(_N // _SUB, _SUB)
    tbl = lax.optimization_barrier(embed_table.reshape(500000, 128))
    tbl = tbl.reshape(1000000, _D)
    out = pl.kernel(
        _emb_body,
        out_type=jax.ShapeDtypeStruct((_N, _D), jnp.float32),
        mesh=mesh,
        scratch_types=[
            pltpu.VMEM((2, _NSUB, _SUB), jnp.int32),
            pltpu.VMEM((2, _C, _D), jnp.float32),
            pltpu.VMEM((_S, _D), jnp.float32),
            pltpu.SemaphoreType.DMA,
            pltpu.SemaphoreType.DMA,
        ],
        compiler_params=pltpu.CompilerParams(
            use_tc_tiling_on_sc=False, needs_layout_passes=False
        ),
    )(x2d, tbl, pos_table)
    return out.reshape(_S, _B, _D)


def kernel(x, embed_table, pos_table):
    return _run(x, embed_table, pos_table)
